# baseline (speedup 1.0000x reference)
import functools

import jax
import jax.numpy as jnp
from jax import lax
from jax.experimental import pallas as pl
from jax.experimental.pallas import tpu as pltpu
from jax.experimental.pallas import tpu_sc as plsc

N = 10000
E = 320000
D = 128
DE = 236
DEP = 240
NPW = 320
NPAD = 32 * NPW
K = 48
NBUF = 3
G = K // 16
NB = 10
TN = N // NB


def _sc_segment_sum(edges, srcs, starts):
    mesh = plsc.VectorSubcoreMesh(core_axis_name="c", subcore_axis_name="s")

    @functools.partial(
        pl.kernel,
        mesh=mesh,
        out_type=jax.ShapeDtypeStruct((NPAD, DEP), jnp.float32),
        scratch_types=[
            pltpu.VMEM((NBUF * K, DE), jnp.float32),
            pltpu.VMEM((NBUF * K,), jnp.int32),
            pltpu.VMEM((48,), jnp.int32),
            pltpu.VMEM((NPW, DEP), jnp.float32),
            pltpu.SemaphoreType.DMA,
            pltpu.SemaphoreType.DMA,
            pltpu.SemaphoreType.DMA,
        ],
        compiler_params=pltpu.CompilerParams(needs_layout_passes=False),
    )
    def seg(edges_hbm, srcs_hbm, starts_hbm, out_hbm, buf_v, id_v,
            st_v, blk_v, sem0, sem1, sem2):
        wid = lax.axis_index("s") * 2 + lax.axis_index("c")
        nlo = pl.multiple_of(wid * NPW, 8)
        ci = lax.iota(jnp.int32, 16)
        zeros16 = jnp.zeros((16,), jnp.float32)

        def zrow(r, carry):
            for v in range(DEP // 16):
                blk_v[r, pl.ds(16 * v, 16)] = zeros16
            return carry
        lax.fori_loop(0, NPW, zrow, 0)

        pltpu.sync_copy(starts_hbm, st_v)

        def pick(j):
            vec = plsc.load_gather(st_v, [jnp.full((16,), 0, jnp.int32) + j])
            return vec[0]

        elo = pick(wid)
        ehi = pick(wid + 1)
        elo8 = elo & ~jnp.int32(7)

        NV = 15
        COFF = [16 * v for v in range(14)] + [220]

        HALVES = (range(0, 8), range(8, NV))

        def flush(prev, acc, vr=range(NV)):
            @pl.when((prev >= nlo) & (prev < nlo + NPW))
            def _():
                prow = prev - nlo
                for v in vr:
                    blk_v[prow, pl.ds(COFF[v], 16)] = acc[v]

        sems = (sem0, sem1, sem2)

        def chunk_e0(g):
            return pl.multiple_of(
                jnp.minimum(elo8 + g * K, E - K), 8)

        def start_dma(g, b):
            e0d = chunk_e0(g)
            pltpu.async_copy(edges_hbm.at[pl.ds(e0d, K)],
                             buf_v.at[pl.ds(b * K, K)], sems[b])
            pltpu.async_copy(srcs_hbm.at[pl.ds(e0d, K)],
                             id_v.at[pl.ds(b * K, K)], sems[b])

        def wait_dma(b):
            pltpu.make_async_copy(edges_hbm.at[pl.ds(0, K)],
                                  buf_v.at[pl.ds(b * K, K)], sems[b]).wait()
            pltpu.make_async_copy(srcs_hbm.at[pl.ds(0, K)],
                                  id_v.at[pl.ds(b * K, K)], sems[b]).wait()

        nch = (ehi - elo8 + (K - 1)) // K
        nchp = jnp.maximum(((nch + NBUF - 1) // NBUF) * NBUF, NBUF)

        lane15 = jnp.full((16,), 15, jnp.int32)
        rollix = (ci + 15) & 15

        def process(g, b, carry):
            e0 = elo8 + g * K
            e0d = chunk_e0(g)
            boff = e0 - e0d + b * K

            def group(gi, c2):
                prev = c2[0]
                pids = c2[1]
                acc = list(c2[2:])
                off = boff + 16 * gi
                ids_raw = id_v[pl.ds(pl.multiple_of(off, 8), 16)]
                evv = (e0 + 16 * gi) + ci
                validv = (evv >= elo) & (evv < ehi)
                idv = jnp.where(validv, ids_raw, -1)
                rolled = idv.at[rollix].get(mode="promise_in_bounds")
                plast = pids.at[lane15].get(mode="promise_in_bounds")
                shifted = jnp.where(ci == 0, plast, rolled)
                bnd = jnp.where(idv != shifted, 1, 0)
                for lane in range(16):
                    lv = jnp.full((16,), lane, jnp.int32)
                    rst = bnd.at[lv].get(mode="promise_in_bounds") != 0
                    idl = idv[lane]
                    flush(jnp.where(idl != prev, prev, -1), acc)
                    erow = e0 + 16 * gi + lane - e0d + b * K
                    for v in range(NV):
                        x = jnp.maximum(buf_v[erow, pl.ds(COFF[v], 16)], 0.0)
                        acc[v] = jnp.where(rst, 0.0, acc[v]) + x
                    prev = idl
                return (prev, idv, *acc)

            return lax.fori_loop(0, G, group, carry)

        start_dma(0, 0)
        start_dma(1, 1)

        def ring(g3, carry):
            for b in range(NBUF):
                g = NBUF * g3 + b
                wait_dma(b)

                @pl.when(g + 2 < nchp)
                def _():
                    start_dma(g + 2, (b + 2) % NBUF)
                carry = process(g, b, carry)
            return carry

        zero_acc = tuple(jnp.zeros((16,), jnp.float32) for _ in range(NV))
        fin = lax.fori_loop(
            0, nchp // NBUF, ring,
            (jnp.int32(-1), jnp.full((16,), -1, jnp.int32), *zero_acc))
        flush(fin[0], list(fin[2:]))

        pltpu.sync_copy(blk_v.at[pl.ds(0, NPW)], out_hbm.at[pl.ds(nlo, NPW)])

    return seg(edges, srcs, starts)


def _tc_a(agg, x, wep, be, w1):
    def body(agg_ref, x_ref, wep_ref, be_ref, w1_ref, z_ref, st_ref):
        i = pl.program_id(0)

        @pl.when(i == 0)
        def _init():
            st_ref[...] = jnp.zeros_like(st_ref)

        a = agg_ref[...]
        xv = x_ref[...]
        for c in range(3):
            h = xv + jnp.dot(a, wep_ref[c], preferred_element_type=jnp.float32)
            h = h + be_ref[pl.ds(c, 1), :]
            z = jnp.dot(h, w1_ref[c], preferred_element_type=jnp.float32)
            z_ref[:, pl.ds(c * D, D)] = z
            st_ref[pl.ds(c, 1), :] += jnp.sum(z, axis=0, keepdims=True)
            st_ref[pl.ds(3 + c, 1), :] += jnp.sum(z * z, axis=0, keepdims=True)

    return pl.pallas_call(
        body,
        grid=(NB,),
        in_specs=[
            pl.BlockSpec((TN, DEP), lambda i: (i, 0)),
            pl.BlockSpec((TN, D), lambda i: (i, 0)),
            pl.BlockSpec((3, DEP, D), lambda i: (0, 0, 0)),
            pl.BlockSpec((3, D), lambda i: (0, 0)),
            pl.BlockSpec((3, D, D), lambda i: (0, 0, 0)),
        ],
        out_specs=[
            pl.BlockSpec((TN, 3 * D), lambda i: (i, 0)),
            pl.BlockSpec((8, D), lambda i: (0, 0)),
        ],
        out_shape=[
            jax.ShapeDtypeStruct((N, 3 * D), jnp.float32),
            jax.ShapeDtypeStruct((8, D), jnp.float32),
        ],
        compiler_params=pltpu.CompilerParams(
            dimension_semantics=("arbitrary",)),
    )(agg, x, wep, be, w1)


def _tc_b(z, stats, gb, w2, lin1_b, lin2_w, lin2_b):
    def body(z_ref, st_ref, gb_ref, w2_ref, l1b_ref, l2w_ref,
             l2b_ref, o_ref):
        acc = jnp.zeros((TN, D), jnp.float32)
        inv_n = jnp.float32(1.0 / N)
        for c in range(3):
            mean = st_ref[pl.ds(c, 1), :] * inv_n
            var = st_ref[pl.ds(3 + c, 1), :] * inv_n - mean * mean
            rstd = lax.rsqrt(var + 1e-5)
            scale = gb_ref[pl.ds(c, 1), :] * rstd
            shift = gb_ref[pl.ds(3 + c, 1), :] - mean * scale
            zc = z_ref[:, pl.ds(c * D, D)]
            y = jnp.maximum(zc * scale + shift, 0.0)
            acc = acc + jnp.dot(y, w2_ref[c],
                                preferred_element_type=jnp.float32)
        hmm = jnp.maximum(acc + l1b_ref[...], 0.0)
        o_ref[...] = (jnp.dot(hmm, l2w_ref[...],
                              preferred_element_type=jnp.float32)
                      + l2b_ref[...])

    return pl.pallas_call(
        body,
        grid=(NB,),
        in_specs=[
            pl.BlockSpec((TN, 3 * D), lambda i: (i, 0)),
            pl.BlockSpec((8, D), lambda i: (0, 0)),
            pl.BlockSpec((8, D), lambda i: (0, 0)),
            pl.BlockSpec((3, D, D), lambda i: (0, 0, 0)),
            pl.BlockSpec((1, D), lambda i: (0, 0)),
            pl.BlockSpec((D, D), lambda i: (0, 0)),
            pl.BlockSpec((1, D), lambda i: (0, 0)),
        ],
        out_specs=pl.BlockSpec((TN, D), lambda i: (i, 0)),
        out_shape=jax.ShapeDtypeStruct((N, D), jnp.float32),
        compiler_params=pltpu.CompilerParams(
            dimension_semantics=("arbitrary",)),
    )(z, stats, gb, w2, lin1_b, lin2_w, lin2_b)


def kernel(paths_srcs, path_tgt_edges_per_src, srcs,
           We1, be1, W1_1, W2_1, gamma1, beta1,
           We2, be2, W1_2, W2_2, gamma2, beta2,
           We3, be3, W1_3, W2_3, gamma3, beta3,
           lin1_W, lin1_b, lin2_W, lin2_b):
    srcs = srcs.astype(jnp.int32)
    bounds = (jnp.arange(33, dtype=jnp.int32) * NPW).astype(srcs.dtype)
    starts = jnp.searchsorted(srcs, bounds).astype(jnp.int32)
    starts = jnp.concatenate(
        [starts, jnp.full((15,), E, jnp.int32)])

    agg = _sc_segment_sum(path_tgt_edges_per_src, srcs, starts)

    wep = jnp.stack([jnp.pad(w, ((0, DEP - DE), (0, 0)))
                     for w in (We1, We2, We3)])
    be = jnp.stack([be1, be2, be3])
    w1 = jnp.stack([W1_1, W1_2, W1_3])
    w2 = jnp.stack([W2_1, W2_2, W2_3])
    gb = jnp.concatenate([
        jnp.stack([gamma1, gamma2, gamma3]),
        jnp.stack([beta1, beta2, beta3]),
        jnp.zeros((2, D), jnp.float32)])

    l1 = jnp.stack([lin1_W[c * D:(c + 1) * D] for c in range(3)])
    w2l1 = jnp.einsum("cde,cef->cdf", w2, l1)

    z, stats = _tc_a(agg, paths_srcs, wep, be, w1)
    out = _tc_b(z, stats, gb, w2l1,
                jnp.reshape(lin1_b, (1, D)),
                lin2_W, jnp.reshape(lin2_b, (1, D)))
    return out


# speedup vs baseline: 1.1152x; 1.1152x over previous
import functools

import jax
import jax.numpy as jnp
from jax import lax
from jax.experimental import pallas as pl
from jax.experimental.pallas import tpu as pltpu
from jax.experimental.pallas import tpu_sc as plsc

N = 10000
E = 320000
D = 128
DE = 236
DEP = 240
NPW = 320
NPAD = 32 * NPW
K = 80
NBUF = 2
G = K // 16
NB = 10
TN = N // NB


def _sc_segment_sum(edges, srcs, starts):
    mesh = plsc.VectorSubcoreMesh(core_axis_name="c", subcore_axis_name="s")

    @functools.partial(
        pl.kernel,
        mesh=mesh,
        out_type=jax.ShapeDtypeStruct((NPAD, DEP), jnp.float32),
        scratch_types=[
            pltpu.VMEM((NBUF * K, DE), jnp.float32),
            pltpu.VMEM((NBUF * K,), jnp.int32),
            pltpu.VMEM((48,), jnp.int32),
            pltpu.VMEM((NPW, DEP), jnp.float32),
            pltpu.SemaphoreType.DMA,
            pltpu.SemaphoreType.DMA,
        ],
        compiler_params=pltpu.CompilerParams(needs_layout_passes=False),
    )
    def seg(edges_hbm, srcs_hbm, starts_hbm, out_hbm, buf_v, id_v,
            st_v, blk_v, sem0, sem1):
        wid = lax.axis_index("s") * 2 + lax.axis_index("c")
        nlo = pl.multiple_of(wid * NPW, 8)
        ci = lax.iota(jnp.int32, 16)
        zeros16 = jnp.zeros((16,), jnp.float32)

        def zrow(r, carry):
            for v in range(DEP // 16):
                blk_v[r, pl.ds(16 * v, 16)] = zeros16
            return carry
        lax.fori_loop(0, NPW, zrow, 0)

        pltpu.sync_copy(starts_hbm, st_v)

        def pick(j):
            vec = plsc.load_gather(st_v, [jnp.full((16,), 0, jnp.int32) + j])
            return vec[0]

        elo = pick(wid)
        ehi = pick(wid + 1)
        elo8 = elo & ~jnp.int32(7)

        NV = 15
        COFF = [16 * v for v in range(14)] + [220]

        HALVES = (range(0, 8), range(8, NV))

        def flush(prev, acc, vr=range(NV)):
            @pl.when((prev >= nlo) & (prev < nlo + NPW))
            def _():
                prow = prev - nlo
                for v in vr:
                    blk_v[prow, pl.ds(COFF[v], 16)] = acc[v]

        sems = (sem0, sem1)

        def chunk_e0(g):
            return pl.multiple_of(
                jnp.minimum(elo8 + g * K, E - K), 8)

        def start_dma(g, b):
            e0d = chunk_e0(g)
            pltpu.async_copy(edges_hbm.at[pl.ds(e0d, K)],
                             buf_v.at[pl.ds(b * K, K)], sems[b])
            pltpu.async_copy(srcs_hbm.at[pl.ds(e0d, K)],
                             id_v.at[pl.ds(b * K, K)], sems[b])

        def wait_dma(b):
            pltpu.make_async_copy(edges_hbm.at[pl.ds(0, K)],
                                  buf_v.at[pl.ds(b * K, K)], sems[b]).wait()
            pltpu.make_async_copy(srcs_hbm.at[pl.ds(0, K)],
                                  id_v.at[pl.ds(b * K, K)], sems[b]).wait()

        nch = (ehi - elo8 + (K - 1)) // K
        nchp = jnp.maximum(((nch + NBUF - 1) // NBUF) * NBUF, NBUF)

        lane15 = jnp.full((16,), 15, jnp.int32)
        rollix = (ci + 15) & 15

        def process(g, b, carry):
            e0 = elo8 + g * K
            e0d = chunk_e0(g)
            boff = e0 - e0d + b * K

            def group(gi, c2):
                prev = c2[0]
                pids = c2[1]
                acc = list(c2[2:])
                off = boff + 16 * gi
                ids_raw = id_v[pl.ds(pl.multiple_of(off, 8), 16)]
                evv = (e0 + 16 * gi) + ci
                validv = (evv >= elo) & (evv < ehi)
                idv = jnp.where(validv, ids_raw, -1)
                rolled = idv.at[rollix].get(mode="promise_in_bounds")
                plast = pids.at[lane15].get(mode="promise_in_bounds")
                shifted = jnp.where(ci == 0, plast, rolled)
                bnd = jnp.where(idv != shifted, 1, 0)
                for lane in range(16):
                    lv = jnp.full((16,), lane, jnp.int32)
                    rst = bnd.at[lv].get(mode="promise_in_bounds") != 0
                    idl = idv[lane]
                    flush(jnp.where(idl != prev, prev, -1), acc)
                    erow = e0 + 16 * gi + lane - e0d + b * K
                    for v in range(NV):
                        x = jnp.maximum(buf_v[erow, pl.ds(COFF[v], 16)], 0.0)
                        acc[v] = jnp.where(rst, 0.0, acc[v]) + x
                    prev = idl
                return (prev, idv, *acc)

            return lax.fori_loop(0, G, group, carry)

        start_dma(0, 0)

        def ring(g3, carry):
            for b in range(NBUF):
                g = NBUF * g3 + b
                wait_dma(b)

                @pl.when(g + 1 < nchp)
                def _():
                    start_dma(g + 1, 1 - b)
                carry = process(g, b, carry)
            return carry

        zero_acc = tuple(jnp.zeros((16,), jnp.float32) for _ in range(NV))
        fin = lax.fori_loop(
            0, nchp // NBUF, ring,
            (jnp.int32(-1), jnp.full((16,), -1, jnp.int32), *zero_acc))
        flush(fin[0], list(fin[2:]))

        pltpu.sync_copy(blk_v.at[pl.ds(0, NPW)], out_hbm.at[pl.ds(nlo, NPW)])

    return seg(edges, srcs, starts)


def _tc_a(agg, x, wep, be, w1):
    def body(agg_ref, x_ref, wep_ref, be_ref, w1_ref, z_ref, st_ref):
        i = pl.program_id(0)

        @pl.when(i == 0)
        def _init():
            st_ref[...] = jnp.zeros_like(st_ref)

        a = agg_ref[...]
        xv = x_ref[...]
        for c in range(3):
            h = xv + jnp.dot(a, wep_ref[c], preferred_element_type=jnp.float32)
            h = h + be_ref[pl.ds(c, 1), :]
            z = jnp.dot(h, w1_ref[c], preferred_element_type=jnp.float32)
            z_ref[:, pl.ds(c * D, D)] = z
            st_ref[pl.ds(c, 1), :] += jnp.sum(z, axis=0, keepdims=True)
            st_ref[pl.ds(3 + c, 1), :] += jnp.sum(z * z, axis=0, keepdims=True)

    return pl.pallas_call(
        body,
        grid=(NB,),
        in_specs=[
            pl.BlockSpec((TN, DEP), lambda i: (i, 0)),
            pl.BlockSpec((TN, D), lambda i: (i, 0)),
            pl.BlockSpec((3, DEP, D), lambda i: (0, 0, 0)),
            pl.BlockSpec((3, D), lambda i: (0, 0)),
            pl.BlockSpec((3, D, D), lambda i: (0, 0, 0)),
        ],
        out_specs=[
            pl.BlockSpec((TN, 3 * D), lambda i: (i, 0)),
            pl.BlockSpec((8, D), lambda i: (0, 0)),
        ],
        out_shape=[
            jax.ShapeDtypeStruct((N, 3 * D), jnp.float32),
            jax.ShapeDtypeStruct((8, D), jnp.float32),
        ],
        compiler_params=pltpu.CompilerParams(
            dimension_semantics=("arbitrary",)),
    )(agg, x, wep, be, w1)


def _tc_b(z, stats, gb, w2, lin1_b, lin2_w, lin2_b):
    def body(z_ref, st_ref, gb_ref, w2_ref, l1b_ref, l2w_ref,
             l2b_ref, o_ref):
        acc = jnp.zeros((TN, D), jnp.float32)
        inv_n = jnp.float32(1.0 / N)
        for c in range(3):
            mean = st_ref[pl.ds(c, 1), :] * inv_n
            var = st_ref[pl.ds(3 + c, 1), :] * inv_n - mean * mean
            rstd = lax.rsqrt(var + 1e-5)
            scale = gb_ref[pl.ds(c, 1), :] * rstd
            shift = gb_ref[pl.ds(3 + c, 1), :] - mean * scale
            zc = z_ref[:, pl.ds(c * D, D)]
            y = jnp.maximum(zc * scale + shift, 0.0)
            acc = acc + jnp.dot(y, w2_ref[c],
                                preferred_element_type=jnp.float32)
        hmm = jnp.maximum(acc + l1b_ref[...], 0.0)
        o_ref[...] = (jnp.dot(hmm, l2w_ref[...],
                              preferred_element_type=jnp.float32)
                      + l2b_ref[...])

    return pl.pallas_call(
        body,
        grid=(NB,),
        in_specs=[
            pl.BlockSpec((TN, 3 * D), lambda i: (i, 0)),
            pl.BlockSpec((8, D), lambda i: (0, 0)),
            pl.BlockSpec((8, D), lambda i: (0, 0)),
            pl.BlockSpec((3, D, D), lambda i: (0, 0, 0)),
            pl.BlockSpec((1, D), lambda i: (0, 0)),
            pl.BlockSpec((D, D), lambda i: (0, 0)),
            pl.BlockSpec((1, D), lambda i: (0, 0)),
        ],
        out_specs=pl.BlockSpec((TN, D), lambda i: (i, 0)),
        out_shape=jax.ShapeDtypeStruct((N, D), jnp.float32),
        compiler_params=pltpu.CompilerParams(
            dimension_semantics=("arbitrary",)),
    )(z, stats, gb, w2, lin1_b, lin2_w, lin2_b)


def kernel(paths_srcs, path_tgt_edges_per_src, srcs,
           We1, be1, W1_1, W2_1, gamma1, beta1,
           We2, be2, W1_2, W2_2, gamma2, beta2,
           We3, be3, W1_3, W2_3, gamma3, beta3,
           lin1_W, lin1_b, lin2_W, lin2_b):
    srcs = srcs.astype(jnp.int32)
    bounds = (jnp.arange(33, dtype=jnp.int32) * NPW).astype(srcs.dtype)
    starts = jnp.searchsorted(srcs, bounds).astype(jnp.int32)
    starts = jnp.concatenate(
        [starts, jnp.full((15,), E, jnp.int32)])

    agg = _sc_segment_sum(path_tgt_edges_per_src, srcs, starts)

    wep = jnp.stack([jnp.pad(w, ((0, DEP - DE), (0, 0)))
                     for w in (We1, We2, We3)])
    be = jnp.stack([be1, be2, be3])
    w1 = jnp.stack([W1_1, W1_2, W1_3])
    w2 = jnp.stack([W2_1, W2_2, W2_3])
    gb = jnp.concatenate([
        jnp.stack([gamma1, gamma2, gamma3]),
        jnp.stack([beta1, beta2, beta3]),
        jnp.zeros((2, D), jnp.float32)])

    l1 = jnp.stack([lin1_W[c * D:(c + 1) * D] for c in range(3)])
    w2l1 = jnp.einsum("cde,cef->cdf", w2, l1)

    z, stats = _tc_a(agg, paths_srcs, wep, be, w1)
    out = _tc_b(z, stats, gb, w2l1,
                jnp.reshape(lin1_b, (1, D)),
                lin2_W, jnp.reshape(lin2_b, (1, D)))
    return out
